# Initial kernel scaffold; baseline (speedup 1.0000x reference)
import jax
import jax.numpy as jnp
from jax.experimental import pallas as pl


def kernel(x, edge_index, edge_weight, W1, b1, W2, b2):
    raise NotImplementedError("write your pallas kernel here")


# speedup vs baseline: 2.5172x; 2.5172x over previous
import jax
import jax.numpy as jnp
from jax.experimental import pallas as pl


N_NODES = 10000
D = 128
_ROWS_PER_BLOCK = 1000


def _mm_body(x_ref, w_ref, o_ref):
    o_ref[...] = jnp.dot(x_ref[...], w_ref[...], preferred_element_type=jnp.float32)


def _matmul(x, W):
    n = x.shape[0]
    grid = n // _ROWS_PER_BLOCK
    return pl.pallas_call(
        _mm_body,
        grid=(grid,),
        in_specs=[
            pl.BlockSpec((_ROWS_PER_BLOCK, D), lambda i: (i, 0)),
            pl.BlockSpec((D, D), lambda i: (0, 0)),
        ],
        out_specs=pl.BlockSpec((_ROWS_PER_BLOCK, D), lambda i: (i, 0)),
        out_shape=jax.ShapeDtypeStruct((n, D), jnp.float32),
    )(x, W)


def _gcn_layer(g, src, dst, ew, dinv, b):
    msg = jnp.take(g, src, axis=0) * ew[:, None]
    scat = jnp.zeros_like(g).at[dst].add(msg)
    return dinv[:, None] * (scat + g) + b


def kernel(x, edge_index, edge_weight, W1, b1, W2, b2):
    src = edge_index[0].astype(jnp.int32)
    dst = edge_index[1].astype(jnp.int32)
    ew = edge_weight

    deg = jnp.ones((N_NODES,), jnp.float32).at[dst].add(ew)
    dinv = jax.lax.rsqrt(deg)

    g1 = dinv[:, None] * _matmul(x, W1)
    x1 = jax.nn.relu(_gcn_layer(g1, src, dst, ew, dinv, b1))
    g2 = dinv[:, None] * _matmul(x1, W2)
    return _gcn_layer(g2, src, dst, ew, dinv, b2)


# speedup vs baseline: 12.3565x; 4.9088x over previous
import dataclasses
import functools

import jax
import jax.numpy as jnp
from jax import lax
from jax.experimental import pallas as pl
from jax.experimental.pallas import tpu as pltpu
from jax.experimental.pallas import tpu_sc as plsc

N_NODES = 10000
N_PAD = 10240
N_EDGES = 320000
D = 128

NC = 2
NS = 16
NW = NC * NS
E_PER_W = N_EDGES // NW
CH = 200
NCHUNK = E_PER_W // CH
ROWS_PER_TILE = N_PAD // NS

_MESH = plsc.VectorSubcoreMesh(core_axis_name="c", subcore_axis_name="s",
                               num_cores=NC, num_subcores=NS)

_ROWS_PER_BLOCK = 1024
_GRID = N_PAD // _ROWS_PER_BLOCK

_SC_PARAMS = pltpu.CompilerParams()
if "needs_layout_passes" in pltpu.CompilerParams.__dataclass_fields__:
    _SC_PARAMS = dataclasses.replace(_SC_PARAMS, needs_layout_passes=False)


def _zero_rows(rows_v, width):
    zero = jnp.zeros((16,), jnp.float32)

    @pl.loop(0, rows_v.shape[0])
    def _(i):
        for f in range(width // 16):
            rows_v[i, pl.ds(16 * f, 16)] = zero


def _zero_acc(rows_v, acc_sh, sid, rows_per_tile):
    base = sid * rows_per_tile
    for ofs in range(0, rows_per_tile, CH):
        n = min(CH, rows_per_tile - ofs)
        pltpu.sync_copy(rows_v.at[pl.ds(0, n)], acc_sh.at[pl.ds(base + ofs, n)])


def _deg_body(dst_hbm, ew_hbm, out_hbm, acc_sh, msg_v, dst_v, ew_v, sem):
    cid = lax.axis_index("c")
    sid = lax.axis_index("s")
    wid = cid * NS + sid

    _zero_rows(msg_v, 16)
    _zero_acc(msg_v, acc_sh, sid, ROWS_PER_TILE)
    plsc.subcore_barrier()

    ebase = wid * E_PER_W

    @pl.loop(0, NCHUNK)
    def _(k):
        cb = ebase + k * CH
        pltpu.sync_copy(dst_hbm.at[pl.ds(cb, CH)], dst_v)
        pltpu.sync_copy(ew_hbm.at[pl.ds(cb, CH)], ew_v)

        @pl.loop(0, CH)
        def _(e):
            w = plsc.load_gather(ew_v, [jnp.full((16,), e, jnp.int32)])
            msg_v[e, pl.ds(0, 16)] = w

        pltpu.async_copy(msg_v, acc_sh.at[dst_v], sem, add=True).wait()

    plsc.subcore_barrier()
    base = sid * ROWS_PER_TILE
    pltpu.sync_copy(acc_sh.at[pl.ds(base, ROWS_PER_TILE)],
                    out_hbm.at[cid].at[pl.ds(base, ROWS_PER_TILE)])


def _sc_degree(dst, ew):
    return pl.kernel(
        _deg_body,
        out_type=jax.ShapeDtypeStruct((NC, N_PAD, 16), jnp.float32),
        mesh=_MESH,
        scratch_types=[
            pltpu.VMEM_SHARED((N_PAD, 16), jnp.float32),
            pltpu.VMEM((CH, 16), jnp.float32),
            pltpu.VMEM((CH,), jnp.int32),
            pltpu.VMEM((CH,), jnp.float32),
            pltpu.SemaphoreType.DMA,
        ],
        compiler_params=_SC_PARAMS,
    )(dst, ew)


def _agg_body(g_hbm, src_hbm, dst_hbm, ew_hbm, out_hbm,
              acc_sh, rows_v, src_v, dst_v, ew_v, sem):
    cid = lax.axis_index("c")
    sid = lax.axis_index("s")
    wid = cid * NS + sid

    _zero_rows(rows_v, D)
    _zero_acc(rows_v, acc_sh, sid, ROWS_PER_TILE)
    plsc.subcore_barrier()

    ebase = wid * E_PER_W

    @pl.loop(0, NCHUNK)
    def _(k):
        cb = ebase + k * CH
        pltpu.sync_copy(src_hbm.at[pl.ds(cb, CH)], src_v)
        pltpu.sync_copy(dst_hbm.at[pl.ds(cb, CH)], dst_v)
        pltpu.sync_copy(ew_hbm.at[pl.ds(cb, CH)], ew_v)
        pltpu.async_copy(g_hbm.at[src_v], rows_v, sem).wait()

        @pl.loop(0, CH)
        def _(e):
            w = plsc.load_gather(ew_v, [jnp.full((16,), e, jnp.int32)])
            for f in range(D // 16):
                sl = pl.ds(16 * f, 16)
                rows_v[e, sl] = rows_v[e, sl] * w

        pltpu.async_copy(rows_v, acc_sh.at[dst_v], sem, add=True).wait()

    plsc.subcore_barrier()
    base = sid * ROWS_PER_TILE
    pltpu.sync_copy(acc_sh.at[pl.ds(base, ROWS_PER_TILE)],
                    out_hbm.at[cid].at[pl.ds(base, ROWS_PER_TILE)])


def _sc_aggregate(g, src, dst, ew):
    return pl.kernel(
        _agg_body,
        out_type=jax.ShapeDtypeStruct((NC, N_PAD, D), jnp.float32),
        mesh=_MESH,
        scratch_types=[
            pltpu.VMEM_SHARED((N_PAD, D), jnp.float32),
            pltpu.VMEM((CH, D), jnp.float32),
            pltpu.VMEM((CH,), jnp.int32),
            pltpu.VMEM((CH,), jnp.int32),
            pltpu.VMEM((CH,), jnp.float32),
            pltpu.SemaphoreType.DMA,
        ],
        compiler_params=_SC_PARAMS,
    )(g, src, dst, ew)


def _mm_rows_spec():
    return pl.BlockSpec((_ROWS_PER_BLOCK, D), lambda i: (i, 0))


def _w_spec():
    return pl.BlockSpec((D, D), lambda i: (0, 0))


def _mm_body(x_ref, w_ref, o_ref):
    o_ref[...] = jnp.dot(x_ref[...], w_ref[...],
                         preferred_element_type=jnp.float32)


def _tc_matmul(x, W):
    return pl.pallas_call(
        _mm_body,
        grid=(_GRID,),
        in_specs=[_mm_rows_spec(), _w_spec()],
        out_specs=_mm_rows_spec(),
        out_shape=jax.ShapeDtypeStruct((N_PAD, D), jnp.float32),
    )(x, W)


def _dinv_g_body(deg_ref, h_ref, g_ref, dinv_ref):
    deg = deg_ref[0, :, 0:1] + deg_ref[1, :, 0:1] + 1.0
    dinv = lax.rsqrt(deg)
    dinv_ref[...] = dinv
    g_ref[...] = dinv * h_ref[...]


def _tc_dinv_g(deg_parts, h1):
    return pl.pallas_call(
        _dinv_g_body,
        grid=(_GRID,),
        in_specs=[
            pl.BlockSpec((NC, _ROWS_PER_BLOCK, 16), lambda i: (0, i, 0)),
            _mm_rows_spec(),
        ],
        out_specs=[
            _mm_rows_spec(),
            pl.BlockSpec((_ROWS_PER_BLOCK, 1), lambda i: (i, 0)),
        ],
        out_shape=[
            jax.ShapeDtypeStruct((N_PAD, D), jnp.float32),
            jax.ShapeDtypeStruct((N_PAD, 1), jnp.float32),
        ],
    )(deg_parts, h1)


def _mid_body(p_ref, g_ref, dinv_ref, b_ref, w_ref, g2_ref):
    s = dinv_ref[...] * (p_ref[0] + p_ref[1] + g_ref[...]) + b_ref[...]
    x1 = jnp.maximum(s, 0.0)
    g2_ref[...] = dinv_ref[...] * jnp.dot(x1, w_ref[...],
                                          preferred_element_type=jnp.float32)


def _tc_mid(parts1, g1, dinv, b1, W2):
    return pl.pallas_call(
        _mid_body,
        grid=(_GRID,),
        in_specs=[
            pl.BlockSpec((NC, _ROWS_PER_BLOCK, D), lambda i: (0, i, 0)),
            _mm_rows_spec(),
            pl.BlockSpec((_ROWS_PER_BLOCK, 1), lambda i: (i, 0)),
            pl.BlockSpec((1, D), lambda i: (0, 0)),
            _w_spec(),
        ],
        out_specs=_mm_rows_spec(),
        out_shape=jax.ShapeDtypeStruct((N_PAD, D), jnp.float32),
    )(parts1, g1, dinv, b1.reshape(1, D), W2)


def _fin_body(p_ref, g_ref, dinv_ref, b_ref, o_ref):
    o_ref[...] = dinv_ref[...] * (p_ref[0] + p_ref[1] + g_ref[...]) + b_ref[...]


def _tc_fin(parts2, g2, dinv, b2):
    return pl.pallas_call(
        _fin_body,
        grid=(_GRID,),
        in_specs=[
            pl.BlockSpec((NC, _ROWS_PER_BLOCK, D), lambda i: (0, i, 0)),
            _mm_rows_spec(),
            pl.BlockSpec((_ROWS_PER_BLOCK, 1), lambda i: (i, 0)),
            pl.BlockSpec((1, D), lambda i: (0, 0)),
        ],
        out_specs=_mm_rows_spec(),
        out_shape=jax.ShapeDtypeStruct((N_PAD, D), jnp.float32),
    )(parts2, g2, dinv, b2.reshape(1, D))


def kernel(x, edge_index, edge_weight, W1, b1, W2, b2):
    src = edge_index[0].astype(jnp.int32)
    dst = edge_index[1].astype(jnp.int32)
    ew = edge_weight.astype(jnp.float32)
    x = jnp.pad(x, ((0, N_PAD - N_NODES), (0, 0)))

    deg_parts = _sc_degree(dst, ew)
    h1 = _tc_matmul(x, W1)
    g1, dinv = _tc_dinv_g(deg_parts, h1)
    parts1 = _sc_aggregate(g1, src, dst, ew)
    g2 = _tc_mid(parts1, g1, dinv, b1, W2)
    parts2 = _sc_aggregate(g2, src, dst, ew)
    out = _tc_fin(parts2, g2, dinv, b2)
    return out[:N_NODES]
